# Initial kernel scaffold; baseline (speedup 1.0000x reference)
import jax
import jax.numpy as jnp
from jax.experimental import pallas as pl


def kernel(x, W_attn, b_attn, adaptive_bias, W_s, b_s):
    raise NotImplementedError("write your pallas kernel here")


# speedup vs baseline: 1.8484x; 1.8484x over previous
import jax
import jax.numpy as jnp
from jax.experimental import pallas as pl
from jax.experimental.pallas import tpu as pltpu

_T = 2048
_D = 1024
_E = 8
_TT = 256


def _fused_kernel(xb_ref, wa_ref, bias_ref, ws_ref, bs_ref, out_ref):
    xb = xb_ref[...]
    x32 = xb.astype(jnp.float32)
    logits = jnp.dot(x32, wa_ref[...], preferred_element_type=jnp.float32)
    logits = logits + bias_ref[...]
    w = jax.nn.softmax(logits, axis=-1)
    acc = jnp.zeros((xb.shape[0], _D), jnp.float32)
    for e in range(_E):
        h = jnp.dot(xb, ws_ref[e], preferred_element_type=jnp.float32)
        h = jnp.tanh(h + bs_ref[e][None, :].astype(jnp.float32))
        acc = acc + w[:, e:e + 1] * h
    out_ref[...] = acc


def kernel(x, W_attn, b_attn, adaptive_bias, W_s, b_s):
    xb = x.astype(jnp.bfloat16)
    wsb = W_s.astype(jnp.bfloat16)
    bias = (b_attn + adaptive_bias).reshape(1, _E)
    grid = (_T // _TT,)
    return pl.pallas_call(
        _fused_kernel,
        grid=grid,
        in_specs=[
            pl.BlockSpec((_TT, _D), lambda t: (t, 0)),
            pl.BlockSpec((_D, _E), lambda t: (0, 0)),
            pl.BlockSpec((1, _E), lambda t: (0, 0)),
            pl.BlockSpec((_E, _D, _D), lambda t: (0, 0, 0)),
            pl.BlockSpec((_E, _D), lambda t: (0, 0)),
        ],
        out_specs=pl.BlockSpec((_TT, _D), lambda t: (t, 0)),
        out_shape=jax.ShapeDtypeStruct((_T, _D), jnp.float32),
        compiler_params=pltpu.CompilerParams(
            dimension_semantics=("arbitrary",),
        ),
    )(xb, W_attn, bias, wsb, b_s)


# speedup vs baseline: 2.3434x; 1.2678x over previous
import jax
import jax.numpy as jnp
from jax.experimental import pallas as pl
from jax.experimental.pallas import tpu as pltpu

_T = 2048
_D = 1024
_E = 8


def _fused_kernel(x_ref, wa_ref, bias_ref, ws_ref, bs_ref, out_ref,
                  xb_ref, w_ref):
    e = pl.program_id(0)

    @pl.when(e == 0)
    def _prologue():
        x32 = x_ref[...]
        logits = jnp.dot(x32, wa_ref[...],
                         preferred_element_type=jnp.float32) + bias_ref[...]
        w_ref[...] = jax.nn.softmax(logits, axis=-1)
        xb_ref[...] = x32.astype(jnp.bfloat16)

    wsb = ws_ref[0].astype(jnp.bfloat16)
    h = jnp.dot(xb_ref[...], wsb, preferred_element_type=jnp.float32)
    h = jnp.tanh(h + bs_ref[0])
    w = w_ref[...]
    lane = jax.lax.broadcasted_iota(jnp.int32, w.shape, 1)
    we = jnp.sum(jnp.where(lane == e, w, 0.0), axis=1, keepdims=True)
    contrib = we * h

    @pl.when(e == 0)
    def _init():
        out_ref[...] = contrib

    @pl.when(e != 0)
    def _acc():
        out_ref[...] = out_ref[...] + contrib


def kernel(x, W_attn, b_attn, adaptive_bias, W_s, b_s):
    bias = (b_attn + adaptive_bias).reshape(1, _E)
    return pl.pallas_call(
        _fused_kernel,
        grid=(_E,),
        in_specs=[
            pl.BlockSpec((_T, _D), lambda e: (0, 0)),
            pl.BlockSpec((_D, _E), lambda e: (0, 0)),
            pl.BlockSpec((1, _E), lambda e: (0, 0)),
            pl.BlockSpec((1, _D, _D), lambda e: (e, 0, 0)),
            pl.BlockSpec((1, 1, _D), lambda e: (e, 0, 0)),
        ],
        out_specs=pl.BlockSpec((_T, _D), lambda e: (0, 0)),
        out_shape=jax.ShapeDtypeStruct((_T, _D), jnp.float32),
        scratch_shapes=[
            pltpu.VMEM((_T, _D), jnp.bfloat16),
            pltpu.VMEM((_T, _E), jnp.float32),
        ],
        compiler_params=pltpu.CompilerParams(
            dimension_semantics=("arbitrary",),
        ),
    )(x, W_attn, bias, W_s, b_s.reshape(_E, 1, _D))
